# baseline (speedup 1.0000x reference)
import jax
import jax.numpy as jnp
from jax import lax
from jax.experimental import pallas as pl
from jax.experimental.pallas import tpu as pltpu
from jax.experimental.pallas import tpu_sc as plsc

_DECAY = 1.0 - 0.01
_MIN_W = 0.01
_N_NODES = 100000
_N_EDGES = 6400000

_NC, _NS, _L = 2, 16, 16
_NW = _NC * _NS

_BLK = 128
_NBLOCKS = _N_EDGES // _BLK
_BPW_LO = _NBLOCKS // _NW
_CB = 13
_CHUNK = _CB * _BLK
_NBUF = 6
_NGROUPS = -(-(-(-(_BPW_LO + 1) // _CB)) // _NBUF)
_NCHUNKS = _NGROUPS * _NBUF


def _sc_body(w_hbm, idx_hbm, mask_hbm, out_hbm,
             mask_v, sd0, w0, sd1, w1, sd2, w2, sd3, w3, sd4, w4, sd5, w5,
             sin0, sin1, sin2, sin3, sin4, sin5,
             sout0, sout1, sout2, sout3, sout4, sout5):
    wid = lax.axis_index("s") * _NC + lax.axis_index("c")
    base_b = wid * _BPW_LO + jnp.minimum(wid, _NW // 2)
    nb = _BPW_LO + jnp.where(wid < _NW // 2, 1, 0)
    bufs = ((sd0, w0, sin0, sout0), (sd1, w1, sin1, sout1),
            (sd2, w2, sin2, sout2), (sd3, w3, sin3, sout3),
            (sd4, w4, sin4, sout4), (sd5, w5, sin5, sout5))

    def chunk_off(ci):
        return (base_b + jnp.minimum(ci * _CB, nb - _CB)) * _BLK

    def start_in(ci, b):
        sd_v, w_v, sem_in, _ = bufs[b]
        off = chunk_off(ci)
        pltpu.async_copy(idx_hbm.at[:, pl.ds(off, _CHUNK)], sd_v, sem_in)
        pltpu.async_copy(w_hbm.at[pl.ds(off, _CHUNK)], w_v, sem_in)

    def wait_in(b):
        sd_v, w_v, sem_in, _ = bufs[b]
        pltpu.make_async_copy(idx_hbm.at[:, pl.ds(0, _CHUNK)], sd_v, sem_in).wait()
        pltpu.make_async_copy(w_hbm.at[pl.ds(0, _CHUNK)], w_v, sem_in).wait()

    def start_out(ci, b):
        _, w_v, _, sem_out = bufs[b]
        pltpu.async_copy(w_v, out_hbm.at[pl.ds(chunk_off(ci), _CHUNK)], sem_out)

    def wait_out(b):
        _, w_v, _, sem_out = bufs[b]
        pltpu.make_async_copy(w_v, out_hbm.at[pl.ds(0, _CHUNK)], sem_out).wait()

    def compute(b):
        sd_v, w_v, _, _ = bufs[b]

        @plsc.parallel_loop(0, _CHUNK, step=_L, unroll=8)
        def _(i):
            sl = pl.ds(i, _L)
            s = plsc.load_gather(mask_v, [sd_v[0, sl]])
            d = plsc.load_gather(mask_v, [sd_v[1, sl]])
            active = (s > 0) & (d > 0)
            w = w_v[sl]
            decayed = jnp.where(active, w, w * _DECAY)
            w_v[sl] = jnp.where(decayed >= _MIN_W, decayed, 0.0)

    pltpu.async_copy(mask_hbm, mask_v, sin0)
    for b in range(_NBUF - 1):
        start_in(b, b)
    pltpu.make_async_copy(mask_hbm, mask_v, sin0).wait()

    def group_body(p, _):
        for b in range(_NBUF):
            ci = _NBUF * p + b
            nbx = (b + _NBUF - 1) % _NBUF
            wait_in(b)
            compute(b)
            start_out(ci, b)
            if b == 0:
                @pl.when(p > 0)
                def _():
                    wait_out(nbx)
                start_in(ci + _NBUF - 1, nbx)
            else:
                @pl.when(p + 1 < _NGROUPS)
                def _():
                    wait_out(nbx)
                    start_in(ci + _NBUF - 1, nbx)
        return 0

    lax.fori_loop(0, _NGROUPS, group_body, 0)
    for b in range(_NBUF):
        wait_out(b)


@jax.jit
def _run(edge_weight, edge_index, activity_mask):
    mesh = plsc.VectorSubcoreMesh(core_axis_name="c", subcore_axis_name="s")
    return pl.kernel(
        _sc_body,
        out_type=jax.ShapeDtypeStruct((_N_EDGES,), jnp.float32),
        mesh=mesh,
        compiler_params=pltpu.CompilerParams(needs_layout_passes=False),
        scratch_types=[
            pltpu.VMEM((_N_NODES,), jnp.int32),
            pltpu.VMEM((2, _CHUNK), jnp.int32),
            pltpu.VMEM((_CHUNK,), jnp.float32),
            pltpu.VMEM((2, _CHUNK), jnp.int32),
            pltpu.VMEM((_CHUNK,), jnp.float32),
            pltpu.VMEM((2, _CHUNK), jnp.int32),
            pltpu.VMEM((_CHUNK,), jnp.float32),
            pltpu.VMEM((2, _CHUNK), jnp.int32),
            pltpu.VMEM((_CHUNK,), jnp.float32),
            pltpu.VMEM((2, _CHUNK), jnp.int32),
            pltpu.VMEM((_CHUNK,), jnp.float32),
            pltpu.VMEM((2, _CHUNK), jnp.int32),
            pltpu.VMEM((_CHUNK,), jnp.float32),
            pltpu.SemaphoreType.DMA,
            pltpu.SemaphoreType.DMA,
            pltpu.SemaphoreType.DMA,
            pltpu.SemaphoreType.DMA,
            pltpu.SemaphoreType.DMA,
            pltpu.SemaphoreType.DMA,
            pltpu.SemaphoreType.DMA,
            pltpu.SemaphoreType.DMA,
            pltpu.SemaphoreType.DMA,
            pltpu.SemaphoreType.DMA,
            pltpu.SemaphoreType.DMA,
            pltpu.SemaphoreType.DMA,
        ],
    )(edge_weight, edge_index, activity_mask)


def kernel(edge_weight, edge_activation, edge_index, activity_mask):
    del edge_activation
    return _run(edge_weight, edge_index, activity_mask)


# speedup vs baseline: 1.0303x; 1.0303x over previous
import jax
import jax.numpy as jnp
from jax import lax
from jax.experimental import pallas as pl
from jax.experimental.pallas import tpu as pltpu
from jax.experimental.pallas import tpu_sc as plsc

_DECAY = 1.0 - 0.01
_MIN_W = 0.01
_N_NODES = 100000
_N_EDGES = 6400000

_NC, _NS, _L = 2, 16, 16
_NW = _NC * _NS

_BLK = 128
_NBLOCKS = _N_EDGES // _BLK
_BPW_LO = _NBLOCKS // _NW
_CB = 16
_CHUNK = _CB * _BLK
_NBUF = 4
_NGROUPS = -(-(-(-(_BPW_LO + 1) // _CB)) // _NBUF)
_NCHUNKS = _NGROUPS * _NBUF


def _sc_body(w_hbm, idx_hbm, mask_hbm, out_hbm,
             mask_v, mask_sh, sd0, w0, sd1, w1, sd2, w2, sd3, w3,
             sin0, sin1, sin2, sin3, sout0, sout1, sout2, sout3):
    wid = lax.axis_index("s") * _NC + lax.axis_index("c")
    base_b = wid * _BPW_LO + jnp.minimum(wid, _NW // 2)
    nb = _BPW_LO + jnp.where(wid < _NW // 2, 1, 0)
    bufs = ((sd0, w0, sin0, sout0), (sd1, w1, sin1, sout1),
            (sd2, w2, sin2, sout2), (sd3, w3, sin3, sout3))

    def chunk_off(ci):
        return (base_b + jnp.minimum(ci * _CB, nb - _CB)) * _BLK

    def start_in(ci, b):
        sd_v, w_v, sem_in, _ = bufs[b]
        off = chunk_off(ci)
        pltpu.async_copy(idx_hbm.at[:, pl.ds(off, _CHUNK)], sd_v, sem_in)
        pltpu.async_copy(w_hbm.at[pl.ds(off, _CHUNK)], w_v, sem_in)

    def wait_in(b):
        sd_v, w_v, sem_in, _ = bufs[b]
        pltpu.make_async_copy(idx_hbm.at[:, pl.ds(0, _CHUNK)], sd_v, sem_in).wait()
        pltpu.make_async_copy(w_hbm.at[pl.ds(0, _CHUNK)], w_v, sem_in).wait()

    def start_out(ci, b):
        _, w_v, _, sem_out = bufs[b]
        pltpu.async_copy(w_v, out_hbm.at[pl.ds(chunk_off(ci), _CHUNK)], sem_out)

    def wait_out(b):
        _, w_v, _, sem_out = bufs[b]
        pltpu.make_async_copy(w_v, out_hbm.at[pl.ds(0, _CHUNK)], sem_out).wait()

    def compute(b):
        sd_v, w_v, _, _ = bufs[b]

        @plsc.parallel_loop(0, _CHUNK, step=_L, unroll=8)
        def _(i):
            sl = pl.ds(i, _L)
            s = plsc.load_gather(mask_v, [sd_v[0, sl]])
            d = plsc.load_gather(mask_v, [sd_v[1, sl]])
            active = (s > 0) & (d > 0)
            w = w_v[sl]
            decayed = jnp.where(active, w, w * _DECAY)
            w_v[sl] = jnp.where(decayed >= _MIN_W, decayed, 0.0)

    for b in range(_NBUF - 1):
        start_in(b, b)

    @pl.when(lax.axis_index("s") == 0)
    def _():
        pltpu.sync_copy(mask_hbm, mask_sh)

    plsc.subcore_barrier()
    pltpu.sync_copy(mask_sh, mask_v)

    def group_body(p, _):
        for b in range(_NBUF):
            ci = _NBUF * p + b
            nbx = (b + _NBUF - 1) % _NBUF
            wait_in(b)
            compute(b)
            start_out(ci, b)
            if b == 0:
                @pl.when(p > 0)
                def _():
                    wait_out(nbx)
                start_in(ci + _NBUF - 1, nbx)
            else:
                @pl.when(p + 1 < _NGROUPS)
                def _():
                    wait_out(nbx)
                    start_in(ci + _NBUF - 1, nbx)
        return 0

    lax.fori_loop(0, _NGROUPS, group_body, 0)
    for b in range(_NBUF):
        wait_out(b)


@jax.jit
def _run(edge_weight, edge_index, activity_mask):
    mesh = plsc.VectorSubcoreMesh(core_axis_name="c", subcore_axis_name="s")
    return pl.kernel(
        _sc_body,
        out_type=jax.ShapeDtypeStruct((_N_EDGES,), jnp.float32),
        mesh=mesh,
        compiler_params=pltpu.CompilerParams(needs_layout_passes=False),
        scratch_types=[
            pltpu.VMEM((_N_NODES,), jnp.int32),
            pltpu.VMEM_SHARED((_N_NODES,), jnp.int32),
            pltpu.VMEM((2, _CHUNK), jnp.int32),
            pltpu.VMEM((_CHUNK,), jnp.float32),
            pltpu.VMEM((2, _CHUNK), jnp.int32),
            pltpu.VMEM((_CHUNK,), jnp.float32),
            pltpu.VMEM((2, _CHUNK), jnp.int32),
            pltpu.VMEM((_CHUNK,), jnp.float32),
            pltpu.VMEM((2, _CHUNK), jnp.int32),
            pltpu.VMEM((_CHUNK,), jnp.float32),
            pltpu.SemaphoreType.DMA,
            pltpu.SemaphoreType.DMA,
            pltpu.SemaphoreType.DMA,
            pltpu.SemaphoreType.DMA,
            pltpu.SemaphoreType.DMA,
            pltpu.SemaphoreType.DMA,
            pltpu.SemaphoreType.DMA,
            pltpu.SemaphoreType.DMA,
        ],
    )(edge_weight, edge_index, activity_mask)


def kernel(edge_weight, edge_activation, edge_index, activity_mask):
    del edge_activation
    return _run(edge_weight, edge_index, activity_mask)


# speedup vs baseline: 1.0848x; 1.0529x over previous
import jax
import jax.numpy as jnp
from jax import lax
from jax.experimental import pallas as pl
from jax.experimental.pallas import tpu as pltpu
from jax.experimental.pallas import tpu_sc as plsc

_DECAY = 1.0 - 0.01
_MIN_W = 0.01
_N_NODES = 100000
_N_EDGES = 6400000

_NC, _NS, _L = 2, 16, 16
_NW = _NC * _NS

_BLK = 128
_NBLOCKS = _N_EDGES // _BLK
_BPW_LO = _NBLOCKS // _NW
_CB = 45
_CHUNK = _CB * _BLK
_NBUF = 5
_NGROUPS = -(-(-(-(_BPW_LO + 1) // _CB)) // _NBUF)
_NCHUNKS = _NGROUPS * _NBUF

_PW = _N_NODES // 4
_PSLICE = 1568
_TSLICE = _PSLICE * 4


def _sc_body(w_hbm, idx_hbm, mask_hbm, out_hbm,
             mask_v, raw_sh, pack_sh, tmp_v,
             sd0, w0, sd1, w1, sd2, w2, sd3, w3, sd4, w4,
             sin0, sin1, sin2, sin3, sin4,
             sout0, sout1, sout2, sout3, sout4):
    wid = lax.axis_index("s") * _NC + lax.axis_index("c")
    sid = lax.axis_index("s")
    base_b = wid * _BPW_LO + jnp.minimum(wid, _NW // 2)
    nb = _BPW_LO + jnp.where(wid < _NW // 2, 1, 0)
    bufs = ((sd0, w0, sin0, sout0), (sd1, w1, sin1, sout1),
            (sd2, w2, sin2, sout2), (sd3, w3, sin3, sout3),
            (sd4, w4, sin4, sout4))

    def chunk_off(ci):
        return (base_b + jnp.minimum(ci * _CB, nb - _CB)) * _BLK

    def start_in(ci, b):
        sd_v, w_v, sem_in, _ = bufs[b]
        off = chunk_off(ci)
        pltpu.async_copy(idx_hbm.at[:, pl.ds(off, _CHUNK)], sd_v, sem_in)
        pltpu.async_copy(w_hbm.at[pl.ds(off, _CHUNK)], w_v, sem_in)

    def wait_in(b):
        sd_v, w_v, sem_in, _ = bufs[b]
        pltpu.make_async_copy(idx_hbm.at[:, pl.ds(0, _CHUNK)], sd_v, sem_in).wait()
        pltpu.make_async_copy(w_hbm.at[pl.ds(0, _CHUNK)], w_v, sem_in).wait()

    def start_out(ci, b):
        _, w_v, _, sem_out = bufs[b]
        pltpu.async_copy(w_v, out_hbm.at[pl.ds(chunk_off(ci), _CHUNK)], sem_out)

    def wait_out(b):
        _, w_v, _, sem_out = bufs[b]
        pltpu.make_async_copy(w_v, out_hbm.at[pl.ds(0, _CHUNK)], sem_out).wait()

    def compute(b):
        sd_v, w_v, _, _ = bufs[b]

        @plsc.parallel_loop(0, _CHUNK, step=_L, unroll=8)
        def _(i):
            sl = pl.ds(i, _L)
            s_idx = sd_v[0, sl]
            d_idx = sd_v[1, sl]
            s_word = plsc.load_gather(mask_v, [s_idx >> 2])
            d_word = plsc.load_gather(mask_v, [d_idx >> 2])
            s_act = (s_word >> ((s_idx & 3) << 3)) & 0xFF
            d_act = (d_word >> ((d_idx & 3) << 3)) & 0xFF
            active = (s_act > 0) & (d_act > 0)
            w = w_v[sl]
            decayed = jnp.where(active, w, w * _DECAY)
            w_v[sl] = jnp.where(decayed >= _MIN_W, decayed, 0.0)

    for b in range(_NBUF - 1):
        start_in(b, b)

    @pl.when(sid == 0)
    def _():
        pltpu.sync_copy(mask_hbm, raw_sh)

    plsc.subcore_barrier()

    pslice_start = jnp.minimum(sid * _PSLICE, _PW - _PSLICE)
    pltpu.sync_copy(raw_sh.at[pl.ds(pslice_start * 4, _TSLICE)], tmp_v)
    lanes4 = jax.lax.iota(jnp.int32, _L) * 4

    @plsc.parallel_loop(0, _PSLICE, step=_L, unroll=4)
    def _(i):
        g0 = plsc.load_gather(tmp_v, [lanes4 + 4 * i + 0])
        g1 = plsc.load_gather(tmp_v, [lanes4 + 4 * i + 1])
        g2 = plsc.load_gather(tmp_v, [lanes4 + 4 * i + 2])
        g3 = plsc.load_gather(tmp_v, [lanes4 + 4 * i + 3])
        mask_v[pl.ds(i, _L)] = g0 | (g1 << 8) | (g2 << 16) | (g3 << 24)

    pltpu.sync_copy(mask_v.at[pl.ds(0, _PSLICE)],
                    pack_sh.at[pl.ds(pslice_start, _PSLICE)])
    plsc.subcore_barrier()
    pltpu.sync_copy(pack_sh, mask_v)

    def group_body(p, _):
        for b in range(_NBUF):
            ci = _NBUF * p + b
            nbx = (b + _NBUF - 1) % _NBUF
            wait_in(b)
            compute(b)
            start_out(ci, b)
            if b == 0:
                @pl.when(p > 0)
                def _():
                    wait_out(nbx)
                start_in(ci + _NBUF - 1, nbx)
            else:
                @pl.when(p + 1 < _NGROUPS)
                def _():
                    wait_out(nbx)
                    start_in(ci + _NBUF - 1, nbx)
        return 0

    lax.fori_loop(0, _NGROUPS, group_body, 0)
    for b in range(_NBUF):
        wait_out(b)


@jax.jit
def _run(edge_weight, edge_index, activity_mask):
    mesh = plsc.VectorSubcoreMesh(core_axis_name="c", subcore_axis_name="s")
    return pl.kernel(
        _sc_body,
        out_type=jax.ShapeDtypeStruct((_N_EDGES,), jnp.float32),
        mesh=mesh,
        compiler_params=pltpu.CompilerParams(needs_layout_passes=False),
        scratch_types=[
            pltpu.VMEM((_PW,), jnp.int32),
            pltpu.VMEM_SHARED((_N_NODES,), jnp.int32),
            pltpu.VMEM_SHARED((_PW,), jnp.int32),
            pltpu.VMEM((_TSLICE,), jnp.int32),
            pltpu.VMEM((2, _CHUNK), jnp.int32),
            pltpu.VMEM((_CHUNK,), jnp.float32),
            pltpu.VMEM((2, _CHUNK), jnp.int32),
            pltpu.VMEM((_CHUNK,), jnp.float32),
            pltpu.VMEM((2, _CHUNK), jnp.int32),
            pltpu.VMEM((_CHUNK,), jnp.float32),
            pltpu.VMEM((2, _CHUNK), jnp.int32),
            pltpu.VMEM((_CHUNK,), jnp.float32),
            pltpu.VMEM((2, _CHUNK), jnp.int32),
            pltpu.VMEM((_CHUNK,), jnp.float32),
            pltpu.SemaphoreType.DMA,
            pltpu.SemaphoreType.DMA,
            pltpu.SemaphoreType.DMA,
            pltpu.SemaphoreType.DMA,
            pltpu.SemaphoreType.DMA,
            pltpu.SemaphoreType.DMA,
            pltpu.SemaphoreType.DMA,
            pltpu.SemaphoreType.DMA,
            pltpu.SemaphoreType.DMA,
            pltpu.SemaphoreType.DMA,
        ],
    )(edge_weight, edge_index, activity_mask)


def kernel(edge_weight, edge_activation, edge_index, activity_mask):
    del edge_activation
    return _run(edge_weight, edge_index, activity_mask)
